# baseline (speedup 1.0000x reference)
import jax
import jax.numpy as jnp
from jax import lax
from jax.experimental import pallas as pl
from jax.experimental.pallas import tpu as pltpu
from jax.experimental.pallas import tpu_sc as plsc

DIM = 64
BATCH = 16384
NC = 2
NS = 16
NW = NC * NS
LANES = 16
B_PER_W = BATCH // NW
G = 32
NCH = B_PER_W // G
NSTREAM = 4


def _extract(vec, q):
    return jnp.squeeze(lax.slice(vec, (q,), (q + 1,)))


def _transe_body(ent, rel, comb, out, idx_v, buf_a, buf_b, acc_v,
                 sem_a, sem_b):
    wid = lax.axis_index("s") * NC + lax.axis_index("c")

    pltpu.sync_copy(comb.at[wid], idx_v)

    def fire(c, buf, sem):
        base = c * (NSTREAM * G)
        for s, tab in enumerate((ent, rel, ent, ent)):
            for sg in range(G // LANES):
                iv = idx_v[pl.ds(base + s * G + sg * LANES, LANES)]
                for q in range(LANES):
                    pltpu.make_async_copy(
                        tab.at[pl.ds(_extract(iv, q), 1)],
                        buf.at[pl.ds(s * G + sg * LANES + q, 1)], sem).start()

    def drain(buf, sem):
        pltpu.make_async_copy(ent.at[pl.ds(0, NSTREAM * G)], buf, sem).wait()

    def compute(buf, acc):
        for q in range(G):
            for k in range(DIM // LANES):
                sl = pl.ds(k * LANES, LANES)
                s_ = buf[q, sl] + buf[G + q, sl]
                dp = s_ - buf[2 * G + q, sl]
                dn = s_ - buf[3 * G + q, sl]
                acc = acc + (dn * dn - dp * dp)
        return acc

    fire(0, buf_a, sem_a)

    def pair(i, acc):
        c = 2 * i
        fire(c + 1, buf_b, sem_b)
        drain(buf_a, sem_a)
        acc = compute(buf_a, acc)

        @pl.when(c + 2 < NCH)
        def _():
            fire(c + 2, buf_a, sem_a)

        drain(buf_b, sem_b)
        acc = compute(buf_b, acc)
        return acc

    acc = lax.fori_loop(0, NCH // 2, pair, jnp.zeros((LANES,), jnp.float32))
    acc_v[...] = acc
    pltpu.sync_copy(acc_v, out.at[pl.ds(wid * LANES, LANES)])


@jax.jit
def _transe_sc(ent, rel, comb):
    mesh = plsc.VectorSubcoreMesh(core_axis_name="c", subcore_axis_name="s")
    grid_kernel = pl.kernel(
        _transe_body,
        out_type=jax.ShapeDtypeStruct((NW * LANES,), jnp.float32),
        mesh=mesh,
        scratch_types=[
            pltpu.VMEM((NCH * NSTREAM * G,), jnp.int32),
            pltpu.VMEM((NSTREAM * G, DIM), jnp.float32),
            pltpu.VMEM((NSTREAM * G, DIM), jnp.float32),
            pltpu.VMEM((LANES,), jnp.float32),
            pltpu.SemaphoreType.DMA,
            pltpu.SemaphoreType.DMA,
        ],
    )
    return grid_kernel(ent, rel, comb)


def kernel(data, entity_embedding_matrix, relation_embedding_matrix):
    idx = data.astype(jnp.int32)
    comb = (idx.reshape(NW, NCH, G, NSTREAM)
               .transpose(0, 1, 3, 2)
               .reshape(NW, NCH * NSTREAM * G))
    partials = _transe_sc(entity_embedding_matrix, relation_embedding_matrix,
                          comb)
    return jnp.sum(partials)
